# Initial kernel scaffold; baseline (speedup 1.0000x reference)
import functools

import jax
import jax.numpy as jnp
from jax import lax
from jax.experimental import pallas as pl
from jax.experimental.pallas import tpu as pltpu
from jax.experimental.pallas import tpu_sc as plsc

N = 10000
E = 320000
D_IN = 128
T_EMB = 32
HID = 128
OUT = 128
N_TYPES = 8
D0 = D_IN + T_EMB

NC = 2
NS = 16
NW = NC * NS
EPW = E // NW
CH = 125
NCH = EPW // CH
RPT = N // NS

_sc_mesh = plsc.VectorSubcoreMesh(core_axis_name="c", subcore_axis_name="s")


@functools.partial(
    pl.kernel,
    mesh=_sc_mesh,
    out_type=jax.ShapeDtypeStruct((NC, N, HID), jnp.float32),
    scratch_types=[
        pltpu.VMEM((NCH, CH), jnp.int32),
        pltpu.VMEM((NCH, CH), jnp.int32),
        pltpu.VMEM((CH, HID), jnp.float32),
        pltpu.VMEM_SHARED((N, HID), jnp.float32),
        pltpu.SemaphoreType.DMA,
    ],
)
def _sc_agg(src_hbm, dst_hbm, m_hbm, zeros_hbm, out_hbm,
            src_v, dst_v, rows_v, acc, sem):
    cid = lax.axis_index("c")
    sid = lax.axis_index("s")
    wid = cid * NS + sid
    pltpu.sync_copy(src_hbm.at[wid], src_v)
    pltpu.sync_copy(dst_hbm.at[wid], dst_v)
    pltpu.sync_copy(zeros_hbm, acc.at[pl.ds(sid * RPT, RPT)])
    plsc.subcore_barrier()

    def body(j, carry):
        pltpu.async_copy(m_hbm.at[src_v.at[j]], rows_v, sem).wait()
        pltpu.sync_copy(rows_v, acc.at[dst_v.at[j]], add=True)
        return carry

    lax.fori_loop(0, NCH, body, 0)
    plsc.subcore_barrier()
    pltpu.sync_copy(acc.at[pl.ds(sid * RPT, RPT)],
                    out_hbm.at[cid, pl.ds(sid * RPT, RPT)])


@functools.partial(
    pl.kernel,
    mesh=_sc_mesh,
    out_type=jax.ShapeDtypeStruct((NC, N, 16), jnp.float32),
    scratch_types=[
        pltpu.VMEM((NCH, CH), jnp.int32),
        pltpu.VMEM((CH, 16), jnp.float32),
        pltpu.VMEM_SHARED((N, 16), jnp.float32),
    ],
)
def _sc_cnt(dst_hbm, ones_hbm, zeros_hbm, out_hbm, dst_v, ones_v, acc):
    cid = lax.axis_index("c")
    sid = lax.axis_index("s")
    wid = cid * NS + sid
    pltpu.sync_copy(dst_hbm.at[wid], dst_v)
    pltpu.sync_copy(ones_hbm, ones_v)
    pltpu.sync_copy(zeros_hbm, acc.at[pl.ds(sid * RPT, RPT)])
    plsc.subcore_barrier()

    def body(j, carry):
        pltpu.sync_copy(ones_v, acc.at[dst_v.at[j]], add=True)
        return carry

    lax.fori_loop(0, NCH, body, 0)
    plsc.subcore_barrier()
    pltpu.sync_copy(acc.at[pl.ds(sid * RPT, RPT)],
                    out_hbm.at[cid, pl.ds(sid * RPT, RPT)])


BLK = 1000


def _tc_prep_body(x_ref, tid_ref, emb_ref, wl_ref, wr_ref, b_ref, m_ref, s_ref):
    x = x_ref[...]
    oh = (tid_ref[...] == lax.broadcasted_iota(jnp.int32, (BLK, N_TYPES), 1)
          ).astype(jnp.float32)
    te = jnp.dot(oh, emb_ref[...], preferred_element_type=jnp.float32)
    wl = wl_ref[...]
    wr = wr_ref[...]
    m_ref[...] = (jnp.dot(x, wl[:D_IN], preferred_element_type=jnp.float32)
                  + jnp.dot(te, wl[D_IN:], preferred_element_type=jnp.float32))
    s_ref[...] = (jnp.dot(x, wr[:D_IN], preferred_element_type=jnp.float32)
                  + jnp.dot(te, wr[D_IN:], preferred_element_type=jnp.float32)
                  + b_ref[...])


def _tc_layer_body(a0_ref, a1_ref, c0_ref, c1_ref, s_ref, wl_ref, wr_ref, b_ref,
                   m_ref, sout_ref):
    cnt = c0_ref[:, 0:1] + c1_ref[:, 0:1]
    inv = 1.0 / jnp.maximum(cnt, 1.0)
    h = (a0_ref[...] + a1_ref[...]) * inv + s_ref[...]
    h = jnp.maximum(h, 0.0)
    m_ref[...] = jnp.dot(h, wl_ref[...], preferred_element_type=jnp.float32)
    sout_ref[...] = (jnp.dot(h, wr_ref[...], preferred_element_type=jnp.float32)
                     + b_ref[...])


def _tc_final_body(a0_ref, a1_ref, c0_ref, c1_ref, s_ref, out_ref):
    cnt = c0_ref[:, 0:1] + c1_ref[:, 0:1]
    inv = 1.0 / jnp.maximum(cnt, 1.0)
    out_ref[...] = (a0_ref[...] + a1_ref[...]) * inv + s_ref[...]


def _row_spec(d):
    return pl.BlockSpec((BLK, d), lambda i: (i, 0))


def _full_spec(r, d):
    return pl.BlockSpec((r, d), lambda i: (0, 0))


def _tc_prep(x, tid2d, emb, wl, wr, b):
    return pl.pallas_call(
        _tc_prep_body,
        grid=(N // BLK,),
        in_specs=[_row_spec(D_IN), pl.BlockSpec((BLK, 1), lambda i: (i, 0)),
                  _full_spec(N_TYPES, T_EMB), _full_spec(D0, HID),
                  _full_spec(D0, HID), _full_spec(1, HID)],
        out_specs=[_row_spec(HID), _row_spec(HID)],
        out_shape=[jax.ShapeDtypeStruct((N, HID), jnp.float32),
                   jax.ShapeDtypeStruct((N, HID), jnp.float32)],
    )(x, tid2d, emb, wl, wr, b)


def _tc_layer(a0, a1, c0, c1, s, wl, wr, b, dout):
    return pl.pallas_call(
        _tc_layer_body,
        grid=(N // BLK,),
        in_specs=[_row_spec(HID), _row_spec(HID), _row_spec(16), _row_spec(16),
                  _row_spec(HID), _full_spec(HID, dout), _full_spec(HID, dout),
                  _full_spec(1, dout)],
        out_specs=[_row_spec(dout), _row_spec(dout)],
        out_shape=[jax.ShapeDtypeStruct((N, dout), jnp.float32),
                   jax.ShapeDtypeStruct((N, dout), jnp.float32)],
    )(a0, a1, c0, c1, s, wl, wr, b)


def _tc_final(a0, a1, c0, c1, s):
    return pl.pallas_call(
        _tc_final_body,
        grid=(N // BLK,),
        in_specs=[_row_spec(OUT), _row_spec(OUT), _row_spec(16), _row_spec(16),
                  _row_spec(OUT)],
        out_specs=_row_spec(OUT),
        out_shape=jax.ShapeDtypeStruct((N, OUT), jnp.float32),
    )(a0, a1, c0, c1, s)


def kernel(x, node_type_ids, edge_index, emb_table,
           W_l1, W_r1, b1, W_l2, W_r2, b2, W_l3, W_r3, b3):
    ei = edge_index.astype(jnp.int32)
    src3d = ei[0].reshape(NW, NCH, CH)
    dst3d = ei[1].reshape(NW, NCH, CH)
    tid2d = node_type_ids.astype(jnp.int32).reshape(N, 1)
    z128 = jnp.zeros((RPT, HID), jnp.float32)
    z16 = jnp.zeros((RPT, 16), jnp.float32)
    ones16 = jnp.ones((CH, 16), jnp.float32)

    cntp = _sc_cnt(dst3d, ones16, z16)
    c0, c1 = cntp[0], cntp[1]

    m1, s1 = _tc_prep(x, tid2d, emb_table, W_l1, W_r1, b1.reshape(1, HID))
    a1p = _sc_agg(src3d, dst3d, m1, z128)
    m2, s2 = _tc_layer(a1p[0], a1p[1], c0, c1, s1, W_l2, W_r2,
                       b2.reshape(1, HID), HID)
    a2p = _sc_agg(src3d, dst3d, m2, z128)
    m3, s3 = _tc_layer(a2p[0], a2p[1], c0, c1, s2, W_l3, W_r3,
                       b3.reshape(1, OUT), OUT)
    a3p = _sc_agg(src3d, dst3d, m3, z128)
    return _tc_final(a3p[0], a3p[1], c0, c1, s3)


# speedup vs baseline: 8.8977x; 8.8977x over previous
import functools

import jax
import jax.numpy as jnp
from jax import lax
from jax.experimental import pallas as pl
from jax.experimental.pallas import tpu as pltpu
from jax.experimental.pallas import tpu_sc as plsc

N = 10000
E = 320000
D_IN = 128
T_EMB = 32
HID = 128
OUT = 128
N_TYPES = 8
D0 = D_IN + T_EMB

NC = 2
NS = 16
NW = NC * NS
EPW = E // NW
CH = 125
NCH = EPW // CH
N_PAD = 10240
RPT = N_PAD // NS

_sc_mesh = plsc.VectorSubcoreMesh(core_axis_name="c", subcore_axis_name="s")


@functools.partial(
    pl.kernel,
    mesh=_sc_mesh,
    out_type=jax.ShapeDtypeStruct((NC, N_PAD, HID), jnp.float32),
    scratch_types=[
        pltpu.VMEM((NCH, CH), jnp.int32),
        pltpu.VMEM((NCH, CH), jnp.int32),
        pltpu.VMEM((CH, HID), jnp.float32),
        pltpu.VMEM_SHARED((N_PAD, HID), jnp.float32),
        pltpu.SemaphoreType.DMA,
    ],
)
def _sc_agg(src_hbm, dst_hbm, m_hbm, zeros_hbm, out_hbm,
            src_v, dst_v, rows_v, acc, sem):
    cid = lax.axis_index("c")
    sid = lax.axis_index("s")
    wid = cid * NS + sid
    pltpu.sync_copy(src_hbm.at[wid], src_v)
    pltpu.sync_copy(dst_hbm.at[wid], dst_v)
    pltpu.sync_copy(zeros_hbm, acc.at[pl.ds(sid * RPT, RPT)])
    plsc.subcore_barrier()

    def body(j, carry):
        pltpu.async_copy(m_hbm.at[src_v.at[j]], rows_v, sem).wait()
        pltpu.sync_copy(rows_v, acc.at[dst_v.at[j]], add=True)
        return carry

    lax.fori_loop(0, NCH, body, 0)
    plsc.subcore_barrier()
    pltpu.sync_copy(acc.at[pl.ds(sid * RPT, RPT)],
                    out_hbm.at[cid, pl.ds(sid * RPT, RPT)])


@functools.partial(
    pl.kernel,
    mesh=_sc_mesh,
    out_type=jax.ShapeDtypeStruct((NC, N_PAD), jnp.float32),
    scratch_types=[
        pltpu.VMEM((EPW,), jnp.int32),
        pltpu.VMEM((N_PAD,), jnp.float32),
        pltpu.VMEM_SHARED((NS, N_PAD), jnp.float32),
        pltpu.VMEM((NS, RPT), jnp.float32),
        pltpu.VMEM((RPT,), jnp.float32),
    ],
    compiler_params=pltpu.CompilerParams(needs_layout_passes=False),
)
def _sc_cnt(dst_hbm, zeros_hbm, out_hbm, dst_v, cnt_v, stage, chunk_v, red_v):
    cid = lax.axis_index("c")
    sid = lax.axis_index("s")
    wid = cid * NS + sid
    pltpu.sync_copy(dst_hbm.at[wid], dst_v)
    pltpu.sync_copy(zeros_hbm, cnt_v)
    ones = jnp.full((16,), 1.0, jnp.float32)

    def body(i, carry):
        idx = dst_v[pl.ds(i * 16, 16)]
        plsc.addupdate_scatter(cnt_v, [idx], ones)
        return carry

    lax.fori_loop(0, EPW // 16, body, 0)
    pltpu.sync_copy(cnt_v, stage.at[sid])
    plsc.subcore_barrier()
    for r in range(NS):
        pltpu.sync_copy(stage.at[r, pl.ds(sid * RPT, RPT)], chunk_v.at[r])

    def rbody(t, carry):
        s = chunk_v[0, pl.ds(t * 16, 16)]
        for r in range(1, NS):
            s = s + chunk_v[r, pl.ds(t * 16, 16)]
        red_v[pl.ds(t * 16, 16)] = s
        return carry

    lax.fori_loop(0, RPT // 16, rbody, 0)
    pltpu.sync_copy(red_v, out_hbm.at[cid, pl.ds(sid * RPT, RPT)])


@functools.partial(
    pl.kernel,
    mesh=_sc_mesh,
    out_type=jax.ShapeDtypeStruct((NC, N_PAD, HID), jnp.float32),
    scratch_types=[
        pltpu.VMEM((NCH, CH), jnp.int32),
        pltpu.VMEM((CH, HID), jnp.float32),
        pltpu.VMEM_SHARED((N_PAD, HID), jnp.float32),
    ],
)
def _sc_cnt_wide(dst_hbm, ones_hbm, zeros_hbm, out_hbm, dst_v, ones_v, acc):
    cid = lax.axis_index("c")
    sid = lax.axis_index("s")
    wid = cid * NS + sid
    pltpu.sync_copy(dst_hbm.at[wid], dst_v)
    pltpu.sync_copy(ones_hbm, ones_v)
    pltpu.sync_copy(zeros_hbm, acc.at[pl.ds(sid * RPT, RPT)])
    plsc.subcore_barrier()

    def body(j, carry):
        pltpu.sync_copy(ones_v, acc.at[dst_v.at[j]], add=True)
        return carry

    lax.fori_loop(0, NCH, body, 0)
    plsc.subcore_barrier()
    pltpu.sync_copy(acc.at[pl.ds(sid * RPT, RPT)],
                    out_hbm.at[cid, pl.ds(sid * RPT, RPT)])


BLK = 1000


def _tc_prep_body(x_ref, tid_ref, emb_ref, wl_ref, wr_ref, b_ref, m_ref, s_ref):
    x = x_ref[...]
    oh = (tid_ref[...] == lax.broadcasted_iota(jnp.int32, (BLK, N_TYPES), 1)
          ).astype(jnp.float32)
    te = jnp.dot(oh, emb_ref[...], preferred_element_type=jnp.float32)
    wl = wl_ref[...]
    wr = wr_ref[...]
    m_ref[...] = (jnp.dot(x, wl[:D_IN], preferred_element_type=jnp.float32)
                  + jnp.dot(te, wl[D_IN:], preferred_element_type=jnp.float32))
    s_ref[...] = (jnp.dot(x, wr[:D_IN], preferred_element_type=jnp.float32)
                  + jnp.dot(te, wr[D_IN:], preferred_element_type=jnp.float32)
                  + b_ref[...])


def _tc_layer_body(a0_ref, a1_ref, c0_ref, c1_ref, s_ref, wl_ref, wr_ref, b_ref,
                   m_ref, sout_ref):
    cnt = c0_ref[...] + c1_ref[...]
    inv = 1.0 / jnp.maximum(cnt, 1.0)
    h = (a0_ref[...] + a1_ref[...]) * inv + s_ref[...]
    h = jnp.maximum(h, 0.0)
    m_ref[...] = jnp.dot(h, wl_ref[...], preferred_element_type=jnp.float32)
    sout_ref[...] = (jnp.dot(h, wr_ref[...], preferred_element_type=jnp.float32)
                     + b_ref[...])


def _tc_final_body(a0_ref, a1_ref, c0_ref, c1_ref, s_ref, out_ref):
    cnt = c0_ref[...] + c1_ref[...]
    inv = 1.0 / jnp.maximum(cnt, 1.0)
    out_ref[...] = (a0_ref[...] + a1_ref[...]) * inv + s_ref[...]


def _row_spec(d):
    return pl.BlockSpec((BLK, d), lambda i: (i, 0))


def _full_spec(r, d):
    return pl.BlockSpec((r, d), lambda i: (0, 0))


def _tc_prep(x, tid2d, emb, wl, wr, b):
    return pl.pallas_call(
        _tc_prep_body,
        grid=(N // BLK,),
        in_specs=[_row_spec(D_IN), pl.BlockSpec((BLK, 1), lambda i: (i, 0)),
                  _full_spec(N_TYPES, T_EMB), _full_spec(D0, HID),
                  _full_spec(D0, HID), _full_spec(1, HID)],
        out_specs=[_row_spec(HID), _row_spec(HID)],
        out_shape=[jax.ShapeDtypeStruct((N, HID), jnp.float32),
                   jax.ShapeDtypeStruct((N, HID), jnp.float32)],
    )(x, tid2d, emb, wl, wr, b)


def _tc_layer(a0, a1, c0, c1, s, wl, wr, b, dout):
    return pl.pallas_call(
        _tc_layer_body,
        grid=(N // BLK,),
        in_specs=[_row_spec(HID), _row_spec(HID), _row_spec(1), _row_spec(1),
                  _row_spec(HID), _full_spec(HID, dout), _full_spec(HID, dout),
                  _full_spec(1, dout)],
        out_specs=[_row_spec(dout), _row_spec(dout)],
        out_shape=[jax.ShapeDtypeStruct((N, dout), jnp.float32),
                   jax.ShapeDtypeStruct((N, dout), jnp.float32)],
    )(a0, a1, c0, c1, s, wl, wr, b)


def _tc_final(a0, a1, c0, c1, s):
    return pl.pallas_call(
        _tc_final_body,
        grid=(N // BLK,),
        in_specs=[_row_spec(OUT), _row_spec(OUT), _row_spec(1), _row_spec(1),
                  _row_spec(OUT)],
        out_specs=_row_spec(OUT),
        out_shape=jax.ShapeDtypeStruct((N, OUT), jnp.float32),
    )(a0, a1, c0, c1, s)


def kernel(x, node_type_ids, edge_index, emb_table,
           W_l1, W_r1, b1, W_l2, W_r2, b2, W_l3, W_r3, b3):
    ei = edge_index.astype(jnp.int32)
    src3d = ei[0].reshape(NW, NCH, CH)
    dst3d = ei[1].reshape(NW, NCH, CH)
    dst2d = ei[1].reshape(NW, EPW)
    tid2d = node_type_ids.astype(jnp.int32).reshape(N, 1)
    z128 = jnp.zeros((RPT, HID), jnp.float32)
    zrow = jnp.zeros((N_PAD,), jnp.float32)

    cntp = _sc_cnt(dst2d, zrow)
    c0 = cntp[0].reshape(N_PAD, 1)
    c1 = cntp[1].reshape(N_PAD, 1)

    m1, s1 = _tc_prep(x, tid2d, emb_table, W_l1, W_r1, b1.reshape(1, HID))
    a1p = _sc_agg(src3d, dst3d, m1, z128)
    m2, s2 = _tc_layer(a1p[0], a1p[1], c0, c1, s1, W_l2, W_r2,
                       b2.reshape(1, HID), HID)
    a2p = _sc_agg(src3d, dst3d, m2, z128)
    m3, s3 = _tc_layer(a2p[0], a2p[1], c0, c1, s2, W_l3, W_r3,
                       b3.reshape(1, OUT), OUT)
    a3p = _sc_agg(src3d, dst3d, m3, z128)
    return _tc_final(a3p[0], a3p[1], c0, c1, s3)


# speedup vs baseline: 13.0250x; 1.4639x over previous
import functools

import jax
import jax.numpy as jnp
from jax import lax
from jax.experimental import pallas as pl
from jax.experimental.pallas import tpu as pltpu
from jax.experimental.pallas import tpu_sc as plsc

N = 10000
E = 320000
D_IN = 128
T_EMB = 32
HID = 128
OUT = 128
N_TYPES = 8
D0 = D_IN + T_EMB

NC = 2
NS = 16
NW = NC * NS
EPW = E // NW
CH = 125
NCH = EPW // CH
N_PAD = 10240
RPT = N_PAD // NS

_sc_mesh = plsc.VectorSubcoreMesh(core_axis_name="c", subcore_axis_name="s")


@functools.partial(
    pl.kernel,
    mesh=_sc_mesh,
    out_type=jax.ShapeDtypeStruct((NC, N_PAD, HID), jnp.float32),
    scratch_types=[
        pltpu.VMEM((NCH, CH), jnp.int32),
        pltpu.VMEM((1, CH), jnp.int32),
        pltpu.VMEM((1, CH), jnp.int32),
        pltpu.VMEM((CH, HID), jnp.float32),
        pltpu.VMEM((CH, HID), jnp.float32),
        pltpu.VMEM_SHARED((N_PAD, HID), jnp.float32),
        pltpu.SemaphoreType.DMA,
        pltpu.SemaphoreType.DMA,
        pltpu.SemaphoreType.DMA,
        pltpu.SemaphoreType.DMA,
    ],
)
def _sc_agg(src_hbm, dstc_hbm, m_hbm, zeros_hbm, out_hbm,
            src_v, dst_a, dst_b, rows_a, rows_b, acc,
            sem_a, sem_b, sem_da, sem_db):
    cid = lax.axis_index("c")
    sid = lax.axis_index("s")
    wid = cid * NS + sid
    pltpu.sync_copy(src_hbm.at[wid], src_v)
    pltpu.sync_copy(zeros_hbm, acc.at[pl.ds(sid * RPT, RPT)])
    plsc.subcore_barrier()

    base = wid * NCH
    pltpu.async_copy(m_hbm.at[src_v.at[0]], rows_a, sem_a)
    pltpu.async_copy(dstc_hbm.at[base], dst_a, sem_da)

    def body(t, carry):
        j = 2 * t
        pltpu.async_copy(m_hbm.at[src_v.at[j + 1]], rows_b, sem_b)
        pltpu.async_copy(dstc_hbm.at[base + j + 1], dst_b, sem_db)
        pltpu.make_async_copy(m_hbm.at[src_v.at[j]], rows_a, sem_a).wait()
        pltpu.make_async_copy(dstc_hbm.at[base], dst_a, sem_da).wait()
        pltpu.sync_copy(rows_a, acc.at[dst_a.at[0]], add=True)

        @pl.when(j + 2 < NCH)
        def _():
            pltpu.async_copy(m_hbm.at[src_v.at[j + 2]], rows_a, sem_a)
            pltpu.async_copy(dstc_hbm.at[base + j + 2], dst_a, sem_da)

        pltpu.make_async_copy(m_hbm.at[src_v.at[j]], rows_b, sem_b).wait()
        pltpu.make_async_copy(dstc_hbm.at[base], dst_b, sem_db).wait()
        pltpu.sync_copy(rows_b, acc.at[dst_b.at[0]], add=True)
        return carry

    lax.fori_loop(0, NCH // 2, body, 0)
    plsc.subcore_barrier()
    pltpu.sync_copy(acc.at[pl.ds(sid * RPT, RPT)],
                    out_hbm.at[cid, pl.ds(sid * RPT, RPT)])


@functools.partial(
    pl.kernel,
    mesh=_sc_mesh,
    out_type=jax.ShapeDtypeStruct((NC, N_PAD), jnp.float32),
    scratch_types=[
        pltpu.VMEM((EPW,), jnp.int32),
        pltpu.VMEM((N_PAD,), jnp.float32),
        pltpu.VMEM_SHARED((NS, N_PAD), jnp.float32),
        pltpu.VMEM((NS, RPT), jnp.float32),
        pltpu.VMEM((RPT,), jnp.float32),
    ],
    compiler_params=pltpu.CompilerParams(needs_layout_passes=False),
)
def _sc_cnt(dst_hbm, zeros_hbm, out_hbm, dst_v, cnt_v, stage, chunk_v, red_v):
    cid = lax.axis_index("c")
    sid = lax.axis_index("s")
    wid = cid * NS + sid
    pltpu.sync_copy(dst_hbm.at[wid], dst_v)
    pltpu.sync_copy(zeros_hbm, cnt_v)
    ones = jnp.full((16,), 1.0, jnp.float32)

    def body(i, carry):
        idx = dst_v[pl.ds(i * 16, 16)]
        plsc.addupdate_scatter(cnt_v, [idx], ones)
        return carry

    lax.fori_loop(0, EPW // 16, body, 0)
    pltpu.sync_copy(cnt_v, stage.at[sid])
    plsc.subcore_barrier()
    for r in range(NS):
        pltpu.sync_copy(stage.at[r, pl.ds(sid * RPT, RPT)], chunk_v.at[r])

    def rbody(t, carry):
        s = chunk_v[0, pl.ds(t * 16, 16)]
        for r in range(1, NS):
            s = s + chunk_v[r, pl.ds(t * 16, 16)]
        red_v[pl.ds(t * 16, 16)] = s
        return carry

    lax.fori_loop(0, RPT // 16, rbody, 0)
    pltpu.sync_copy(red_v, out_hbm.at[cid, pl.ds(sid * RPT, RPT)])


@functools.partial(
    pl.kernel,
    mesh=_sc_mesh,
    out_type=jax.ShapeDtypeStruct((NC, N_PAD, HID), jnp.float32),
    scratch_types=[
        pltpu.VMEM((NCH, CH), jnp.int32),
        pltpu.VMEM((CH, HID), jnp.float32),
        pltpu.VMEM_SHARED((N_PAD, HID), jnp.float32),
    ],
)
def _sc_cnt_wide(dst_hbm, ones_hbm, zeros_hbm, out_hbm, dst_v, ones_v, acc):
    cid = lax.axis_index("c")
    sid = lax.axis_index("s")
    wid = cid * NS + sid
    pltpu.sync_copy(dst_hbm.at[wid], dst_v)
    pltpu.sync_copy(ones_hbm, ones_v)
    pltpu.sync_copy(zeros_hbm, acc.at[pl.ds(sid * RPT, RPT)])
    plsc.subcore_barrier()

    def body(j, carry):
        pltpu.sync_copy(ones_v, acc.at[dst_v.at[j]], add=True)
        return carry

    lax.fori_loop(0, NCH, body, 0)
    plsc.subcore_barrier()
    pltpu.sync_copy(acc.at[pl.ds(sid * RPT, RPT)],
                    out_hbm.at[cid, pl.ds(sid * RPT, RPT)])


BLK = 1000


def _tc_prep_body(x_ref, tid_ref, emb_ref, wl_ref, wr_ref, b_ref, m_ref, s_ref):
    x = x_ref[...]
    oh = (tid_ref[...] == lax.broadcasted_iota(jnp.int32, (BLK, N_TYPES), 1)
          ).astype(jnp.float32)
    te = jnp.dot(oh, emb_ref[...], preferred_element_type=jnp.float32)
    wl = wl_ref[...]
    wr = wr_ref[...]
    m_ref[...] = (jnp.dot(x, wl[:D_IN], preferred_element_type=jnp.float32)
                  + jnp.dot(te, wl[D_IN:], preferred_element_type=jnp.float32))
    s_ref[...] = (jnp.dot(x, wr[:D_IN], preferred_element_type=jnp.float32)
                  + jnp.dot(te, wr[D_IN:], preferred_element_type=jnp.float32)
                  + b_ref[...])


def _tc_layer_body(a0_ref, a1_ref, c0_ref, c1_ref, s_ref, wl_ref, wr_ref, b_ref,
                   m_ref, sout_ref):
    cnt = c0_ref[...] + c1_ref[...]
    inv = 1.0 / jnp.maximum(cnt, 1.0)
    h = (a0_ref[...] + a1_ref[...]) * inv + s_ref[...]
    h = jnp.maximum(h, 0.0)
    m_ref[...] = jnp.dot(h, wl_ref[...], preferred_element_type=jnp.float32)
    sout_ref[...] = (jnp.dot(h, wr_ref[...], preferred_element_type=jnp.float32)
                     + b_ref[...])


def _tc_final_body(a0_ref, a1_ref, c0_ref, c1_ref, s_ref, out_ref):
    cnt = c0_ref[...] + c1_ref[...]
    inv = 1.0 / jnp.maximum(cnt, 1.0)
    out_ref[...] = (a0_ref[...] + a1_ref[...]) * inv + s_ref[...]


def _row_spec(d):
    return pl.BlockSpec((BLK, d), lambda i: (i, 0))


def _full_spec(r, d):
    return pl.BlockSpec((r, d), lambda i: (0, 0))


def _tc_prep(x, tid2d, emb, wl, wr, b):
    return pl.pallas_call(
        _tc_prep_body,
        grid=(N // BLK,),
        in_specs=[_row_spec(D_IN), pl.BlockSpec((BLK, 1), lambda i: (i, 0)),
                  _full_spec(N_TYPES, T_EMB), _full_spec(D0, HID),
                  _full_spec(D0, HID), _full_spec(1, HID)],
        out_specs=[_row_spec(HID), _row_spec(HID)],
        out_shape=[jax.ShapeDtypeStruct((N, HID), jnp.float32),
                   jax.ShapeDtypeStruct((N, HID), jnp.float32)],
    )(x, tid2d, emb, wl, wr, b)


def _tc_layer(a0, a1, c0, c1, s, wl, wr, b, dout):
    return pl.pallas_call(
        _tc_layer_body,
        grid=(N // BLK,),
        in_specs=[_row_spec(HID), _row_spec(HID), _row_spec(1), _row_spec(1),
                  _row_spec(HID), _full_spec(HID, dout), _full_spec(HID, dout),
                  _full_spec(1, dout)],
        out_specs=[_row_spec(dout), _row_spec(dout)],
        out_shape=[jax.ShapeDtypeStruct((N, dout), jnp.float32),
                   jax.ShapeDtypeStruct((N, dout), jnp.float32)],
    )(a0, a1, c0, c1, s, wl, wr, b)


def _tc_final(a0, a1, c0, c1, s):
    return pl.pallas_call(
        _tc_final_body,
        grid=(N // BLK,),
        in_specs=[_row_spec(OUT), _row_spec(OUT), _row_spec(1), _row_spec(1),
                  _row_spec(OUT)],
        out_specs=_row_spec(OUT),
        out_shape=jax.ShapeDtypeStruct((N, OUT), jnp.float32),
    )(a0, a1, c0, c1, s)


def kernel(x, node_type_ids, edge_index, emb_table,
           W_l1, W_r1, b1, W_l2, W_r2, b2, W_l3, W_r3, b3):
    ei = edge_index.astype(jnp.int32)
    src3d = ei[0].reshape(NW, NCH, CH)
    dstc = ei[1].reshape(NW * NCH, 1, CH)
    dst2d = ei[1].reshape(NW, EPW)
    tid2d = node_type_ids.astype(jnp.int32).reshape(N, 1)
    z128 = jnp.zeros((RPT, HID), jnp.float32)
    zrow = jnp.zeros((N_PAD,), jnp.float32)

    cntp = _sc_cnt(dst2d, zrow)
    c0 = cntp[0].reshape(N_PAD, 1)
    c1 = cntp[1].reshape(N_PAD, 1)

    m1, s1 = _tc_prep(x, tid2d, emb_table, W_l1, W_r1, b1.reshape(1, HID))
    a1p = _sc_agg(src3d, dstc, m1, z128)
    m2, s2 = _tc_layer(a1p[0], a1p[1], c0, c1, s1, W_l2, W_r2,
                       b2.reshape(1, HID), HID)
    a2p = _sc_agg(src3d, dstc, m2, z128)
    m3, s3 = _tc_layer(a2p[0], a2p[1], c0, c1, s2, W_l3, W_r3,
                       b3.reshape(1, OUT), OUT)
    a3p = _sc_agg(src3d, dstc, m3, z128)
    return _tc_final(a3p[0], a3p[1], c0, c1, s3)
